# baseline (speedup 1.0000x reference)
import functools

import jax
import jax.numpy as jnp
from jax import lax
from jax.experimental import pallas as pl
from jax.experimental.pallas import tpu as pltpu
from jax.experimental.pallas import tpu_sc as plsc

_EPS = 1e-3
_NW = 32
_CHUNK = 1500
_CPAD = 1504
_U = 32
_MROW = _CPAD * 16 // 128


def _main_body(xin4_ref, a_ref, b_ref, w4_ref, e_ref,
               x4_ref, same_ref, keys_ref, stats_ref):
    nr = xin4_ref.shape[1]
    c4 = xin4_ref.shape[2]
    xin4 = xin4_ref[...].reshape(nr, c4)
    x4_ref[...] = lax.dot_general(
        xin4, w4_ref[...], dimension_numbers=(((1,), (0,)), ((), ())),
        preferred_element_type=jnp.float32).reshape(1, nr, 128)

    m36 = lax.dot_general(
        xin4, xin4, dimension_numbers=(((0,), (0,)), ((), ())),
        preferred_element_type=jnp.float32)
    s36 = jnp.sum(xin4, axis=0, keepdims=True)
    stats_ref[...] = jnp.concatenate([m36, s36], axis=0).reshape(
        1, c4 + 1, c4)

    A = a_ref[...].reshape(_CPAD, 8)
    B = b_ref[...].reshape(_CPAD, 8)
    S = jnp.where((A == B) & (A >= 0), 1.0, 0.0)
    same_ref[...] = lax.dot_general(
        S, e_ref[...], dimension_numbers=(((1,), (0,)), ((), ())),
        preferred_element_type=jnp.float32).reshape(1, _CPAD, 128)

    rows = []
    for c in range(8):
        first = lax.slice(A, (188 * c, 0), (188 * c + 1, 1))
        last = lax.slice(A, (188 * c + 187, 3), (188 * c + 188, 4))
        rows.append(jnp.concatenate(
            [jnp.broadcast_to(first, (1, 16)),
             jnp.broadcast_to(last, (1, 16))], axis=1))
    keys_ref[...] = jnp.concatenate(rows, axis=0).reshape(1, 8, 32)


def _out_body(tin_ref, rep_ref, w_ref, ab_ref, o_ref):
    n = o_ref.shape[2]
    cin = tin_ref.shape[1]
    u = w_ref.shape[0]
    tin = tin_ref[...].reshape(cin, n)
    xt = lax.dot_general(
        w_ref[...], tin, dimension_numbers=(((1,), (0,)), ((), ())),
        preferred_element_type=jnp.float32)
    a = lax.slice(ab_ref[...], (0, 0), (u, 1))
    o = lax.slice(ab_ref[...], (0, 1), (u, 2))
    y = jnp.maximum(xt * a + o, 0.0)
    repT = rep_ref[...].reshape(u, n)
    o_ref[...] = jnp.concatenate([y, repT], axis=0).reshape(1, 2 * u, n)


def _splat(val, dtype=jnp.int32):
    return jnp.full((16,), val, dtype)


def _sc_seg(x_hbm, same_hbm, keys_hbm, ab_hbm, out_hbm,
            x_v, m_v, lm_v, keys_v, ab_v, pub_v, nb_v, vals_sp):
    w = lax.axis_index("c") * 16 + lax.axis_index("s")
    base = w * (_CHUNK * _U)
    pltpu.sync_copy(x_hbm.at[pl.ds(base, _CHUNK * _U)], x_v)
    mbase = w * (_MROW * 128)
    pltpu.sync_copy(same_hbm.at[pl.ds(mbase, _MROW * 128)], m_v)
    pltpu.sync_copy(keys_hbm.at[pl.ds(0, _NW * 32)], keys_v)
    pltpu.sync_copy(ab_hbm.at[pl.ds(0, 64)], ab_v)

    a0 = ab_v[pl.ds(0, 16)]
    a1 = ab_v[pl.ds(16, 16)]
    o0 = ab_v[pl.ds(32, 16)]
    o1 = ab_v[pl.ds(48, 16)]

    one = jnp.ones((16,), jnp.float32)

    y0 = jnp.maximum(x_v[pl.ds(0, 16)] * a0 + o0, 0.0)
    y1 = jnp.maximum(x_v[pl.ds(16, 16)] * a1 + o1, 0.0)
    x_v[pl.ds(0, 16)] = y0
    x_v[pl.ds(16, 16)] = y1
    lm_v[pl.ds(0, 16)] = one

    def fwd(p, carry):
        c0, c1, l0, l1, lm = carry
        m = m_v[pl.ds(p * 16, 16)]
        lm = lm * m
        lm_v[pl.ds(p * 16, 16)] = lm
        off = p * _U
        y0 = jnp.maximum(x_v[pl.ds(off, 16)] * a0 + o0, 0.0)
        y1 = jnp.maximum(x_v[pl.ds(off + 16, 16)] * a1 + o1, 0.0)
        c0 = jnp.maximum(y0, c0 * m)
        c1 = jnp.maximum(y1, c1 * m)
        l0 = jnp.maximum(l0, y0 * lm)
        l1 = jnp.maximum(l1, y1 * lm)
        x_v[pl.ds(off, 16)] = c0
        x_v[pl.ds(off + 16, 16)] = c1
        return c0, c1, l0, l1, lm
    c0, c1, l0, l1, _ = lax.fori_loop(1, _CHUNK, fwd,
                                      (y0, y1, y0, y1, one))

    pub_v[pl.ds(0, 16)] = l0
    pub_v[pl.ds(16, 16)] = l1
    pub_v[pl.ds(32, 16)] = c0
    pub_v[pl.ds(48, 16)] = c1
    pltpu.sync_copy(pub_v, vals_sp.at[pl.ds(w * 64, 64)])
    plsc.subcore_barrier()

    b = w // 8
    wl = w - 8 * b
    pltpu.sync_copy(vals_sp.at[pl.ds(b * 512, 512)], nb_v)

    myoff = wl * 64
    leadG0 = nb_v[pl.ds(myoff, 16)]
    leadG1 = nb_v[pl.ds(myoff + 16, 16)]
    tailG0 = nb_v[pl.ds(myoff + 32, 16)]
    tailG1 = nb_v[pl.ds(myoff + 48, 16)]
    myfirst = keys_v[pl.ds(w * 32, 16)]
    mylast = keys_v[pl.ds(w * 32 + 16, 16)]

    one = jnp.ones((16,), jnp.float32)

    chain = one
    for j in range(1, 8):
        q = wl - j
        validv = jnp.where(_splat(q) >= 0, 1.0, 0.0)
        qc = jnp.maximum(q, 0)
        qg = 8 * b + qc
        qfirst = keys_v[pl.ds(qg * 32, 16)]
        qlast = keys_v[pl.ds(qg * 32 + 16, 16)]
        match = jnp.where(qlast == myfirst, 1.0, 0.0)
        step = chain * match * validv
        leadG0 = jnp.maximum(leadG0, nb_v[pl.ds(qc * 64 + 32, 16)] * step)
        leadG1 = jnp.maximum(leadG1, nb_v[pl.ds(qc * 64 + 48, 16)] * step)
        chain = step * jnp.where(qfirst == qlast, 1.0, 0.0)

    chain = one
    for j in range(1, 8):
        q = wl + j
        validv = jnp.where(_splat(q) <= 7, 1.0, 0.0)
        qc = jnp.minimum(q, 7)
        qg = 8 * b + qc
        qfirst = keys_v[pl.ds(qg * 32, 16)]
        qlast = keys_v[pl.ds(qg * 32 + 16, 16)]
        match = jnp.where(qfirst == mylast, 1.0, 0.0)
        step = chain * match * validv
        tailG0 = jnp.maximum(tailG0, nb_v[pl.ds(qc * 64, 16)] * step)
        tailG1 = jnp.maximum(tailG1, nb_v[pl.ds(qc * 64 + 16, 16)] * step)
        chain = step * jnp.where(qfirst == qlast, 1.0, 0.0)

    lastoff = (_CHUNK - 1) * _U
    lml = lm_v[pl.ds((_CHUNK - 1) * 16, 16)]
    t0 = jnp.maximum(tailG0, leadG0 * lml)
    t1 = jnp.maximum(tailG1, leadG1 * lml)
    x_v[pl.ds(lastoff, 16)] = t0
    x_v[pl.ds(lastoff + 16, 16)] = t1

    def bwd(q, carry):
        t0, t1 = carry
        p = (_CHUNK - 2) - q
        m = m_v[pl.ds((p + 1) * 16, 16)]
        lm = lm_v[pl.ds(p * 16, 16)]
        off = p * _U
        t0 = jnp.maximum(
            jnp.maximum(x_v[pl.ds(off, 16)], t0 * m), leadG0 * lm)
        t1 = jnp.maximum(
            jnp.maximum(x_v[pl.ds(off + 16, 16)], t1 * m), leadG1 * lm)
        x_v[pl.ds(off, 16)] = t0
        x_v[pl.ds(off + 16, 16)] = t1
        return t0, t1
    lax.fori_loop(0, _CHUNK - 1, bwd, (t0, t1))

    pltpu.sync_copy(x_v, out_hbm.at[pl.ds(base, _CHUNK * _U)])


def kernel(inputs, indices, W, gamma, beta):
    bs, n, cin = inputs.shape
    u = W.shape[0]
    npts = bs * n
    cpb = n // _CHUNK
    nr = n * cin // 36
    c4 = 4 * cin

    xin4 = inputs.reshape(bs, nr, c4)
    Wt = jnp.transpose(W)
    wrows = [jnp.concatenate(
        [jnp.zeros((cin, u * k), jnp.float32), Wt,
         jnp.zeros((cin, u * (3 - k)), jnp.float32)], axis=1)
        for k in range(4)]
    W4 = jnp.concatenate(wrows, axis=0)

    idx32 = indices.astype(jnp.int32)
    idxc = jnp.pad(idx32.reshape(_NW, _CHUNK),
                   ((0, 0), (0, _CPAD - _CHUNK)), constant_values=-1)
    flat = idxc.reshape(_NW * _CPAD)
    prev = jnp.concatenate([jnp.full((1,), -1, jnp.int32), flat[:-1]])
    A3 = flat.reshape(bs, _CPAD, 8)
    B3 = prev.reshape(bs, _CPAD, 8)
    E = (jnp.arange(128)[None, :] // 16
         == jnp.arange(8)[:, None]).astype(jnp.float32)

    x4, same_f, keys, stats = pl.pallas_call(
        _main_body,
        grid=(bs,),
        in_specs=[
            pl.BlockSpec((1, nr, c4), lambda b: (b, 0, 0)),
            pl.BlockSpec((1, _CPAD, 8), lambda b: (b, 0, 0)),
            pl.BlockSpec((1, _CPAD, 8), lambda b: (b, 0, 0)),
            pl.BlockSpec((c4, 128), lambda b: (0, 0)),
            pl.BlockSpec((8, 128), lambda b: (0, 0)),
        ],
        out_specs=[
            pl.BlockSpec((1, nr, 128), lambda b: (b, 0, 0)),
            pl.BlockSpec((1, _CPAD, 128), lambda b: (b, 0, 0)),
            pl.BlockSpec((1, cpb, 32), lambda b: (b, 0, 0)),
            pl.BlockSpec((1, c4 + 1, c4), lambda b: (b, 0, 0)),
        ],
        out_shape=(
            jax.ShapeDtypeStruct((bs, nr, 128), jnp.float32),
            jax.ShapeDtypeStruct((bs, _CPAD, 128), jnp.float32),
            jax.ShapeDtypeStruct((bs, cpb, 32), jnp.int32),
            jax.ShapeDtypeStruct((bs, c4 + 1, c4), jnp.float32),
        ),
        compiler_params=pltpu.CompilerParams(
            vmem_limit_bytes=100 * 1024 * 1024),
    )(xin4, A3, B3, W4, E)

    m36 = jnp.sum(stats[:, :c4, :], axis=0)
    s36 = jnp.sum(stats[:, c4, :], axis=0)
    m9 = sum(m36[9 * k:9 * k + 9, 9 * k:9 * k + 9] for k in range(4))
    s9 = sum(s36[9 * k:9 * k + 9] for k in range(4))
    mu9 = s9 / npts
    mean = W @ mu9
    e2 = jnp.sum((W @ (m9 / npts)) * W, axis=1)
    var = e2 - mean * mean
    a32 = gamma * lax.rsqrt(var + _EPS)
    o32 = beta - mean * a32
    ab = jnp.concatenate([a32, o32]).astype(jnp.float32)

    mesh = plsc.VectorSubcoreMesh(core_axis_name="c", subcore_axis_name="s")
    sc = functools.partial(
        pl.kernel, mesh=mesh,
        out_type=jax.ShapeDtypeStruct((npts * u,), jnp.float32),
        scratch_types=[
            pltpu.VMEM((_CHUNK * _U,), jnp.float32),
            pltpu.VMEM((_MROW * 128,), jnp.float32),
            pltpu.VMEM((_CHUNK * 16,), jnp.float32),
            pltpu.VMEM((_NW * 32,), jnp.int32),
            pltpu.VMEM((64,), jnp.float32),
            pltpu.VMEM((64,), jnp.float32),
            pltpu.VMEM((512,), jnp.float32),
            pltpu.VMEM_SHARED((_NW * 64,), jnp.float32),
        ],
    )(_sc_seg)

    rep = sc(x4.reshape(npts * u), same_f.reshape(bs * _CPAD * 128),
             keys.reshape(_NW * 32), ab)

    tin = jnp.transpose(inputs, (0, 2, 1))
    out_t = pl.pallas_call(
        _out_body,
        grid=(bs,),
        in_specs=[
            pl.BlockSpec((1, cin, n), lambda b: (b, 0, 0)),
            pl.BlockSpec((1, u, n), lambda b: (b, 0, 0)),
            pl.BlockSpec((u, cin), lambda b: (0, 0)),
            pl.BlockSpec((u, 2), lambda b: (0, 0)),
        ],
        out_specs=pl.BlockSpec((1, 2 * u, n), lambda b: (b, 0, 0)),
        out_shape=jax.ShapeDtypeStruct((bs, 2 * u, n), jnp.float32),
        compiler_params=pltpu.CompilerParams(
            vmem_limit_bytes=100 * 1024 * 1024),
    )(tin, jnp.transpose(rep.reshape(bs, n, u), (0, 2, 1)), W,
      jnp.stack([a32, o32], axis=1).astype(jnp.float32))

    return jnp.transpose(out_t, (0, 2, 1))


# speedup vs baseline: 1.0473x; 1.0473x over previous
import functools

import jax
import jax.numpy as jnp
from jax import lax
from jax.experimental import pallas as pl
from jax.experimental.pallas import tpu as pltpu
from jax.experimental.pallas import tpu_sc as plsc

_EPS = 1e-3
_NW = 32
_CHUNK = 1500
_CPAD = 1504
_U = 32
_MROW = _CPAD * 16 // 128


def _main_body(xin4_ref, a_ref, b_ref, w4_ref, e_ref,
               x4_ref, same_ref, keys_ref, stats_ref):
    nr = xin4_ref.shape[1]
    c4 = xin4_ref.shape[2]
    xin4 = xin4_ref[...].reshape(nr, c4)
    x4_ref[...] = lax.dot_general(
        xin4, w4_ref[...], dimension_numbers=(((1,), (0,)), ((), ())),
        preferred_element_type=jnp.float32).reshape(1, nr, 128)

    m36 = lax.dot_general(
        xin4, xin4, dimension_numbers=(((0,), (0,)), ((), ())),
        preferred_element_type=jnp.float32)
    s36 = jnp.sum(xin4, axis=0, keepdims=True)
    stats_ref[...] = jnp.concatenate([m36, s36], axis=0).reshape(
        1, c4 + 1, c4)

    A = a_ref[...].reshape(_CPAD, 8)
    B = b_ref[...].reshape(_CPAD, 8)
    S = jnp.where((A == B) & (A >= 0), 1.0, 0.0)
    same_ref[...] = lax.dot_general(
        S, e_ref[...], dimension_numbers=(((1,), (0,)), ((), ())),
        preferred_element_type=jnp.float32).reshape(1, _CPAD, 128)

    rows = []
    for c in range(8):
        first = lax.slice(A, (188 * c, 0), (188 * c + 1, 1))
        last = lax.slice(A, (188 * c + 187, 3), (188 * c + 188, 4))
        rows.append(jnp.concatenate(
            [jnp.broadcast_to(first, (1, 16)),
             jnp.broadcast_to(last, (1, 16))], axis=1))
    keys_ref[...] = jnp.concatenate(rows, axis=0).reshape(1, 8, 32)


def _out_body(tin_ref, rep_ref, w_ref, ab_ref, o_ref):
    n = o_ref.shape[2]
    cin = tin_ref.shape[1]
    u = w_ref.shape[0]
    tin = tin_ref[...].reshape(cin, n)
    xt = lax.dot_general(
        w_ref[...], tin, dimension_numbers=(((1,), (0,)), ((), ())),
        preferred_element_type=jnp.float32)
    a = lax.slice(ab_ref[...], (0, 0), (u, 1))
    o = lax.slice(ab_ref[...], (0, 1), (u, 2))
    y = jnp.maximum(xt * a + o, 0.0)
    repT = jnp.transpose(rep_ref[...].reshape(n, u))
    o_ref[...] = jnp.concatenate([y, repT], axis=0).reshape(1, 2 * u, n)


def _splat(val, dtype=jnp.int32):
    return jnp.full((16,), val, dtype)


def _sc_seg(x_hbm, same_hbm, keys_hbm, ab_hbm, out_hbm,
            x_v, m_v, lm_v, keys_v, ab_v, pub_v, nb_v, vals_sp):
    w = lax.axis_index("c") * 16 + lax.axis_index("s")
    base = w * (_CHUNK * _U)
    pltpu.sync_copy(x_hbm.at[pl.ds(base, _CHUNK * _U)], x_v)
    mbase = w * (_MROW * 128)
    pltpu.sync_copy(same_hbm.at[pl.ds(mbase, _MROW * 128)], m_v)
    pltpu.sync_copy(keys_hbm.at[pl.ds(0, _NW * 32)], keys_v)
    pltpu.sync_copy(ab_hbm.at[pl.ds(0, 64)], ab_v)

    a0 = ab_v[pl.ds(0, 16)]
    a1 = ab_v[pl.ds(16, 16)]
    o0 = ab_v[pl.ds(32, 16)]
    o1 = ab_v[pl.ds(48, 16)]

    one = jnp.ones((16,), jnp.float32)

    y0 = jnp.maximum(x_v[pl.ds(0, 16)] * a0 + o0, 0.0)
    y1 = jnp.maximum(x_v[pl.ds(16, 16)] * a1 + o1, 0.0)
    x_v[pl.ds(0, 16)] = y0
    x_v[pl.ds(16, 16)] = y1
    lm_v[pl.ds(0, 16)] = one

    def fwd_one(p, carry):
        c0, c1, l0, l1, lm = carry
        m = m_v[pl.ds(p * 16, 16)]
        lm = lm * m
        lm_v[pl.ds(p * 16, 16)] = lm
        off = p * _U
        y0 = jnp.maximum(x_v[pl.ds(off, 16)] * a0 + o0, 0.0)
        y1 = jnp.maximum(x_v[pl.ds(off + 16, 16)] * a1 + o1, 0.0)
        c0 = jnp.maximum(y0, c0 * m)
        c1 = jnp.maximum(y1, c1 * m)
        l0 = jnp.maximum(l0, y0 * lm)
        l1 = jnp.maximum(l1, y1 * lm)
        x_v[pl.ds(off, 16)] = c0
        x_v[pl.ds(off + 16, 16)] = c1
        return c0, c1, l0, l1, lm

    def fwd2(i, carry):
        p = 1 + 2 * i
        return fwd_one(p + 1, fwd_one(p, carry))
    st = lax.fori_loop(0, (_CHUNK - 2) // 2, fwd2, (y0, y1, y0, y1, one))
    c0, c1, l0, l1, _ = fwd_one(_CHUNK - 1, st)

    pub_v[pl.ds(0, 16)] = l0
    pub_v[pl.ds(16, 16)] = l1
    pub_v[pl.ds(32, 16)] = c0
    pub_v[pl.ds(48, 16)] = c1
    pltpu.sync_copy(pub_v, vals_sp.at[pl.ds(w * 64, 64)])
    plsc.subcore_barrier()

    b = w // 8
    wl = w - 8 * b
    pltpu.sync_copy(vals_sp.at[pl.ds(b * 512, 512)], nb_v)

    myoff = wl * 64
    leadG0 = nb_v[pl.ds(myoff, 16)]
    leadG1 = nb_v[pl.ds(myoff + 16, 16)]
    tailG0 = nb_v[pl.ds(myoff + 32, 16)]
    tailG1 = nb_v[pl.ds(myoff + 48, 16)]
    myfirst = keys_v[pl.ds(w * 32, 16)]
    mylast = keys_v[pl.ds(w * 32 + 16, 16)]

    one = jnp.ones((16,), jnp.float32)

    chain = one
    for j in range(1, 8):
        q = wl - j
        validv = jnp.where(_splat(q) >= 0, 1.0, 0.0)
        qc = jnp.maximum(q, 0)
        qg = 8 * b + qc
        qfirst = keys_v[pl.ds(qg * 32, 16)]
        qlast = keys_v[pl.ds(qg * 32 + 16, 16)]
        match = jnp.where(qlast == myfirst, 1.0, 0.0)
        step = chain * match * validv
        leadG0 = jnp.maximum(leadG0, nb_v[pl.ds(qc * 64 + 32, 16)] * step)
        leadG1 = jnp.maximum(leadG1, nb_v[pl.ds(qc * 64 + 48, 16)] * step)
        chain = step * jnp.where(qfirst == qlast, 1.0, 0.0)

    chain = one
    for j in range(1, 8):
        q = wl + j
        validv = jnp.where(_splat(q) <= 7, 1.0, 0.0)
        qc = jnp.minimum(q, 7)
        qg = 8 * b + qc
        qfirst = keys_v[pl.ds(qg * 32, 16)]
        qlast = keys_v[pl.ds(qg * 32 + 16, 16)]
        match = jnp.where(qfirst == mylast, 1.0, 0.0)
        step = chain * match * validv
        tailG0 = jnp.maximum(tailG0, nb_v[pl.ds(qc * 64, 16)] * step)
        tailG1 = jnp.maximum(tailG1, nb_v[pl.ds(qc * 64 + 16, 16)] * step)
        chain = step * jnp.where(qfirst == qlast, 1.0, 0.0)

    lastoff = (_CHUNK - 1) * _U
    lml = lm_v[pl.ds((_CHUNK - 1) * 16, 16)]
    t0 = jnp.maximum(tailG0, leadG0 * lml)
    t1 = jnp.maximum(tailG1, leadG1 * lml)
    x_v[pl.ds(lastoff, 16)] = t0
    x_v[pl.ds(lastoff + 16, 16)] = t1

    def bwd_one(p, carry):
        t0, t1 = carry
        m = m_v[pl.ds((p + 1) * 16, 16)]
        lm = lm_v[pl.ds(p * 16, 16)]
        off = p * _U
        t0 = jnp.maximum(
            jnp.maximum(x_v[pl.ds(off, 16)], t0 * m), leadG0 * lm)
        t1 = jnp.maximum(
            jnp.maximum(x_v[pl.ds(off + 16, 16)], t1 * m), leadG1 * lm)
        x_v[pl.ds(off, 16)] = t0
        x_v[pl.ds(off + 16, 16)] = t1
        return t0, t1

    def bwd2(q, carry):
        p = (_CHUNK - 2) - 2 * q
        return bwd_one(p - 1, bwd_one(p, carry))
    st2 = lax.fori_loop(0, (_CHUNK - 2) // 2, bwd2, (t0, t1))
    bwd_one(0, st2)

    pltpu.sync_copy(x_v, out_hbm.at[pl.ds(base, _CHUNK * _U)])


def kernel(inputs, indices, W, gamma, beta):
    bs, n, cin = inputs.shape
    u = W.shape[0]
    npts = bs * n
    cpb = n // _CHUNK
    nr = n * cin // 36
    c4 = 4 * cin

    xin4 = inputs.reshape(bs, nr, c4)
    Wt = jnp.transpose(W)
    wrows = [jnp.concatenate(
        [jnp.zeros((cin, u * k), jnp.float32), Wt,
         jnp.zeros((cin, u * (3 - k)), jnp.float32)], axis=1)
        for k in range(4)]
    W4 = jnp.concatenate(wrows, axis=0)

    idx32 = indices.astype(jnp.int32)
    idxc = jnp.pad(idx32.reshape(_NW, _CHUNK),
                   ((0, 0), (0, _CPAD - _CHUNK)), constant_values=-1)
    flat = idxc.reshape(_NW * _CPAD)
    prev = jnp.concatenate([jnp.full((1,), -1, jnp.int32), flat[:-1]])
    A3 = flat.reshape(bs, _CPAD, 8)
    B3 = prev.reshape(bs, _CPAD, 8)
    E = (jnp.arange(128)[None, :] // 16
         == jnp.arange(8)[:, None]).astype(jnp.float32)

    x4, same_f, keys, stats = pl.pallas_call(
        _main_body,
        grid=(bs,),
        in_specs=[
            pl.BlockSpec((1, nr, c4), lambda b: (b, 0, 0)),
            pl.BlockSpec((1, _CPAD, 8), lambda b: (b, 0, 0)),
            pl.BlockSpec((1, _CPAD, 8), lambda b: (b, 0, 0)),
            pl.BlockSpec((c4, 128), lambda b: (0, 0)),
            pl.BlockSpec((8, 128), lambda b: (0, 0)),
        ],
        out_specs=[
            pl.BlockSpec((1, nr, 128), lambda b: (b, 0, 0)),
            pl.BlockSpec((1, _CPAD, 128), lambda b: (b, 0, 0)),
            pl.BlockSpec((1, cpb, 32), lambda b: (b, 0, 0)),
            pl.BlockSpec((1, c4 + 1, c4), lambda b: (b, 0, 0)),
        ],
        out_shape=(
            jax.ShapeDtypeStruct((bs, nr, 128), jnp.float32),
            jax.ShapeDtypeStruct((bs, _CPAD, 128), jnp.float32),
            jax.ShapeDtypeStruct((bs, cpb, 32), jnp.int32),
            jax.ShapeDtypeStruct((bs, c4 + 1, c4), jnp.float32),
        ),
        compiler_params=pltpu.CompilerParams(
            vmem_limit_bytes=100 * 1024 * 1024),
    )(xin4, A3, B3, W4, E)

    m36 = jnp.sum(stats[:, :c4, :], axis=0)
    s36 = jnp.sum(stats[:, c4, :], axis=0)
    m9 = sum(m36[9 * k:9 * k + 9, 9 * k:9 * k + 9] for k in range(4))
    s9 = sum(s36[9 * k:9 * k + 9] for k in range(4))
    mu9 = s9 / npts
    mean = W @ mu9
    e2 = jnp.sum((W @ (m9 / npts)) * W, axis=1)
    var = e2 - mean * mean
    a32 = gamma * lax.rsqrt(var + _EPS)
    o32 = beta - mean * a32
    ab = jnp.concatenate([a32, o32]).astype(jnp.float32)

    mesh = plsc.VectorSubcoreMesh(core_axis_name="c", subcore_axis_name="s")
    sc = functools.partial(
        pl.kernel, mesh=mesh,
        out_type=jax.ShapeDtypeStruct((npts * u,), jnp.float32),
        scratch_types=[
            pltpu.VMEM((_CHUNK * _U,), jnp.float32),
            pltpu.VMEM((_MROW * 128,), jnp.float32),
            pltpu.VMEM((_CHUNK * 16,), jnp.float32),
            pltpu.VMEM((_NW * 32,), jnp.int32),
            pltpu.VMEM((64,), jnp.float32),
            pltpu.VMEM((64,), jnp.float32),
            pltpu.VMEM((512,), jnp.float32),
            pltpu.VMEM_SHARED((_NW * 64,), jnp.float32),
        ],
    )(_sc_seg)

    rep = sc(x4.reshape(npts * u), same_f.reshape(bs * _CPAD * 128),
             keys.reshape(_NW * 32), ab)

    tin = jnp.transpose(inputs, (0, 2, 1))
    out_t = pl.pallas_call(
        _out_body,
        grid=(bs,),
        in_specs=[
            pl.BlockSpec((1, cin, n), lambda b: (b, 0, 0)),
            pl.BlockSpec((1, n, u), lambda b: (b, 0, 0)),
            pl.BlockSpec((u, cin), lambda b: (0, 0)),
            pl.BlockSpec((u, 2), lambda b: (0, 0)),
        ],
        out_specs=pl.BlockSpec((1, 2 * u, n), lambda b: (b, 0, 0)),
        out_shape=jax.ShapeDtypeStruct((bs, 2 * u, n), jnp.float32),
        compiler_params=pltpu.CompilerParams(
            vmem_limit_bytes=100 * 1024 * 1024),
    )(tin, rep.reshape(bs, n, u), W,
      jnp.stack([a32, o32], axis=1).astype(jnp.float32))

    return jnp.transpose(out_t, (0, 2, 1))
